# baseline (speedup 1.0000x reference)
import functools

import jax
import jax.numpy as jnp
from jax import lax
from jax.experimental import pallas as pl
from jax.experimental.pallas import tpu as pltpu
from jax.experimental.pallas import tpu_sc as plsc
from jax._src.pallas import mpmd as _mpmd


def _scatter_body(S, U, rows_per_w, NC, cache_hbm, update_hbm,
                  pos_hbm, out_hbm, upd_v, idx_v, pos_v, sem):
    del cache_hbm
    w = lax.axis_index("s") * NC + lax.axis_index("c")
    base = w * rows_per_w
    pltpu.sync_copy(pos_hbm, pos_v)
    pltpu.sync_copy(update_hbm.at[pl.ds(base, rows_per_w)], upd_v)
    posv = pos_v[...]
    lane = lax.iota(jnp.int32, 16)
    ub = U.bit_length() - 1
    sb = S.bit_length() - 1
    rel = ((lane >> ub) << sb) + (posv + (lane & (U - 1))) % S
    for c in range(rows_per_w // 16):
        idx_v[pl.ds(c * 16, 16)] = (((base + c * 16) >> ub) << sb) + rel
    pltpu.async_copy(upd_v, out_hbm.at[idx_v], sem).wait()


def kernel(cache, update, pos):
    B, H, S, D = cache.shape
    U = update.shape[-2]
    n_rows = B * H * U
    NW = 32
    rows_per_w = n_rows // NW

    cache_flat = cache.reshape(B * H * S, D)
    update_flat = update.reshape(n_rows, D)
    pos_arr = jnp.broadcast_to(jnp.asarray(pos, jnp.int32), (16,))

    mesh = plsc.VectorSubcoreMesh(core_axis_name="c", subcore_axis_name="s")
    NC = 2
    body = functools.partial(_scatter_body, S, U, rows_per_w, NC)
    run = _mpmd._mpmd_map(
        [(mesh, body)],
        jax.ShapeDtypeStruct((B * H * S, D), cache.dtype),
        input_output_aliases={0: 0},
        scratch_types=[
            pltpu.VMEM((rows_per_w, D), jnp.float32),
            pltpu.VMEM((rows_per_w,), jnp.int32),
            pltpu.VMEM((16,), jnp.int32),
            pltpu.SemaphoreType.DMA,
        ],
        name="kvcache_scatter_sc",
    )
    out = run(cache_flat, update_flat, pos_arr)
    return out.reshape(B, H, S, D)
